# Initial kernel scaffold; baseline (speedup 1.0000x reference)
import jax
import jax.numpy as jnp
from jax.experimental import pallas as pl


def kernel(x, pos, batch, W0_0, b0_0, W0_1, b0_1, W0_2, b0_2, W1_0, b1_0, W1_1, b1_1, W1_2, b1_2, W2_0, b2_0, W2_1, b2_1, W2_2, b2_2):
    raise NotImplementedError("write your pallas kernel here")


# speedup vs baseline: 5.4527x; 5.4527x over previous
import functools

import jax
import jax.numpy as jnp
import numpy as np
from jax import lax
from jax.experimental import pallas as pl
from jax.experimental.pallas import tpu as pltpu
from jax.experimental.pallas import tpu_sc as plsc

N_PTS = 8192
N_Q = 2048
KMAX = 64
D_TAB = 128
_R_LIST = (0.2, 0.4, 0.8)
_K_LIST = (16, 32, 64)


def _fps_body(px_ref, py_ref, pz_ref, pb_ref, qx_ref, qy_ref, qz_ref, qb_ref):
    px = px_ref[...]
    py = py_ref[...]
    pz = pz_ref[...]
    pb = pb_ref[...]
    shape = px.shape
    ri = lax.broadcasted_iota(jnp.int32, shape, 0)
    ci = lax.broadcasted_iota(jnp.int32, shape, 1)
    flat = ri * 128 + ci
    flatf = flat.astype(jnp.float32)
    qshape = (16, 128)
    qri = lax.broadcasted_iota(jnp.int32, qshape, 0)
    qci = lax.broadcasted_iota(jnp.int32, qshape, 1)
    qflat = qri * 128 + qci

    def body(i, carry):
        cur, min_d, qx, qy, qz, qb = carry
        onehot = flat == cur
        cx = jnp.sum(jnp.where(onehot, px, 0.0))
        cy = jnp.sum(jnp.where(onehot, py, 0.0))
        cz = jnp.sum(jnp.where(onehot, pz, 0.0))
        cb = jnp.sum(jnp.where(onehot, pb, 0))
        qsel = qflat == (i - 1)
        qx = jnp.where(qsel, cx, qx)
        qy = jnp.where(qsel, cy, qy)
        qz = jnp.where(qsel, cz, qz)
        qb = jnp.where(qsel, cb, qb)
        d = (px - cx) ** 2 + (py - cy) ** 2 + (pz - cz) ** 2
        d = jnp.where(pb == cb, d, jnp.inf)
        min_d = jnp.minimum(min_d, d)
        m = jnp.max(min_d)
        nxt = jnp.min(jnp.where(min_d == m, flatf, 1e9)).astype(jnp.int32)
        return nxt, min_d, qx, qy, qz, qb

    init = (
        jnp.int32(0),
        jnp.full(shape, jnp.inf, jnp.float32),
        jnp.zeros(qshape, jnp.float32),
        jnp.zeros(qshape, jnp.float32),
        jnp.zeros(qshape, jnp.float32),
        jnp.zeros(qshape, jnp.int32),
    )
    _, _, qx, qy, qz, qb = lax.fori_loop(1, N_Q + 1, body, init)
    qx_ref[...] = qx
    qy_ref[...] = qy
    qz_ref[...] = qz
    qb_ref[...] = qb


def _fps(px, py, pz, pb):
    f = jax.ShapeDtypeStruct((16, 128), jnp.float32)
    i = jax.ShapeDtypeStruct((16, 128), jnp.int32)
    return pl.pallas_call(
        _fps_body,
        out_shape=(f, f, f, i),
    )(px, py, pz, pb)


def _knn_body(q_ref, px_ref, py_ref, pz_ref, pb_ref, nbr_ref, d2_ref):
    q = q_ref[...]
    qx = q[:, 0:1]
    qy = q[:, 1:2]
    qz = q[:, 2:3]
    qb = q[:, 3:4]
    px = px_ref[...]
    py = py_ref[...]
    pz = pz_ref[...]
    pb = pb_ref[...]
    d2 = (qx - px) ** 2 + (qy - py) ** 2 + (qz - pz) ** 2
    d2m = jnp.where(pb == qb, d2, jnp.inf)
    lane = lax.broadcasted_iota(jnp.int32, d2m.shape, 1).astype(jnp.float32)
    col = lax.broadcasted_iota(jnp.int32, (8, KMAX), 1)

    def body(k, carry):
        d2m, nbr, d2v = carry
        m = jnp.min(d2m, axis=1, keepdims=True)
        idx = jnp.min(jnp.where(d2m == m, lane, 1e9), axis=1, keepdims=True)
        d2m = jnp.where(lane == idx, jnp.inf, d2m)
        sel = col == k
        nbr = jnp.where(sel, idx.astype(jnp.int32), nbr)
        d2v = jnp.where(sel, m, d2v)
        return d2m, nbr, d2v

    init = (d2m, jnp.zeros((8, KMAX), jnp.int32), jnp.zeros((8, KMAX), jnp.float32))
    _, nbr, d2v = lax.fori_loop(0, KMAX, body, init)
    nbr_ref[...] = nbr
    d2_ref[...] = d2v


def _knn(qall, px, py, pz, pb):
    grid = (N_Q // 8,)
    return pl.pallas_call(
        _knn_body,
        grid=grid,
        in_specs=[
            pl.BlockSpec((8, 4), lambda i: (i, 0)),
            pl.BlockSpec((8, N_PTS), lambda i: (0, 0)),
            pl.BlockSpec((8, N_PTS), lambda i: (0, 0)),
            pl.BlockSpec((8, N_PTS), lambda i: (0, 0)),
            pl.BlockSpec((8, N_PTS), lambda i: (0, 0)),
        ],
        out_specs=[
            pl.BlockSpec((8, KMAX), lambda i: (i, 0)),
            pl.BlockSpec((8, KMAX), lambda i: (i, 0)),
        ],
        out_shape=[
            jax.ShapeDtypeStruct((N_Q, KMAX), jnp.int32),
            jax.ShapeDtypeStruct((N_Q, KMAX), jnp.float32),
        ],
    )(qall, px, py, pz, pb)


_SC_NW = 32
_SC_CHUNK = 128
_ROWS_PER_W = (N_Q * KMAX) // _SC_NW


def _sc_gather(table, idx_flat):
    mesh = plsc.VectorSubcoreMesh(core_axis_name="c", subcore_axis_name="s")

    @functools.partial(
        pl.kernel,
        mesh=mesh,
        out_type=jax.ShapeDtypeStruct((N_Q * KMAX, D_TAB), jnp.float32),
        scratch_types=[
            pltpu.VMEM((_SC_CHUNK,), jnp.int32),
            pltpu.VMEM((_SC_CHUNK, D_TAB), jnp.float32),
            pltpu.SemaphoreType.DMA,
        ],
    )
    def k(table_hbm, idx_hbm, out_hbm, idx_v, rows_v, sem):
        wid = lax.axis_index("s") * 2 + lax.axis_index("c")
        wbase = wid * _ROWS_PER_W

        def body(c, carry):
            base = wbase + c * _SC_CHUNK
            pltpu.sync_copy(idx_hbm.at[pl.ds(base, _SC_CHUNK)], idx_v)
            pltpu.async_copy(table_hbm.at[idx_v], rows_v, sem).wait()
            pltpu.sync_copy(rows_v, out_hbm.at[pl.ds(base, _SC_CHUNK)])
            return carry

        lax.fori_loop(0, _ROWS_PER_W // _SC_CHUNK, body, 0)

    return k(table, idx_flat)


def _conv_body(K, C3, r2, g_ref, q_ref, d2_ref, w0_ref, b0_ref, w1_ref, b1_ref,
               w2_ref, b2_ref, out_ref, a1_ref):
    QB = q_ref.shape[0]
    q = q_ref[...]
    qx = q[:, 0:1]
    qy = q[:, 1:2]
    qz = q[:, 2:3]
    w0 = w0_ref[...]
    qcorr = (qx * w0_ref[64:65, :] + qy * w0_ref[65:66, :]
             + qz * w0_ref[66:67, :])
    b0 = b0_ref[...]
    for k in range(K):
        g = g_ref[k]
        h = jnp.dot(g, w0, preferred_element_type=jnp.float32) + b0 - qcorr
        a1_ref[k * QB:(k + 1) * QB, :] = jnp.maximum(h, 0.0)
    a1 = a1_ref[...]
    a2 = jnp.maximum(
        jnp.dot(a1, w1_ref[...], preferred_element_type=jnp.float32)
        + b1_ref[...], 0.0)
    a3 = jnp.maximum(
        jnp.dot(a2, w2_ref[...], preferred_element_type=jnp.float32)
        + b2_ref[...], 0.0)
    acc = jnp.full((QB, C3), -1.0, jnp.float32)
    for k in range(K):
        mask = d2_ref[:, k:k + 1] <= r2
        acc = jnp.maximum(acc, jnp.where(mask, a3[k * QB:(k + 1) * QB, :], -1.0))
    out_ref[...] = jnp.where(acc < 0.0, 0.0, acc)


def _conv(g3, qall, d2v, params, K, r):
    (w0p, b0), (w1, b1), (w2, b2) = params
    C1 = w0p.shape[1]
    C3 = w2.shape[1]
    QB = 128
    r2 = np.float32(r * r)
    grid = (N_Q // QB,)
    body = functools.partial(_conv_body, K, C3, r2)
    return pl.pallas_call(
        body,
        grid=grid,
        in_specs=[
            pl.BlockSpec((K, QB, D_TAB), lambda i: (0, i, 0)),
            pl.BlockSpec((QB, 4), lambda i: (i, 0)),
            pl.BlockSpec((QB, KMAX), lambda i: (i, 0)),
            pl.BlockSpec(w0p.shape, lambda i: (0, 0)),
            pl.BlockSpec(b0.shape, lambda i: (0, 0)),
            pl.BlockSpec(w1.shape, lambda i: (0, 0)),
            pl.BlockSpec(b1.shape, lambda i: (0, 0)),
            pl.BlockSpec(w2.shape, lambda i: (0, 0)),
            pl.BlockSpec(b2.shape, lambda i: (0, 0)),
        ],
        out_specs=pl.BlockSpec((QB, C3), lambda i: (i, 0)),
        out_shape=jax.ShapeDtypeStruct((N_Q, C3), jnp.float32),
        scratch_shapes=[pltpu.VMEM((K * QB, C1), jnp.float32)],
    )(g3, qall, d2v, w0p, b0, w1, b1, w2, b2)


def kernel(x, pos, batch, W0_0, b0_0, W0_1, b0_1, W0_2, b0_2,
           W1_0, b1_0, W1_1, b1_1, W1_2, b1_2,
           W2_0, b2_0, W2_1, b2_1, W2_2, b2_2):
    px = pos[:, 0].reshape(64, 128)
    py = pos[:, 1].reshape(64, 128)
    pz = pos[:, 2].reshape(64, 128)
    pb = batch.astype(jnp.int32).reshape(64, 128)

    qx, qy, qz, qb = _fps(px, py, pz, pb)
    qxf = qx.reshape(N_Q)
    qyf = qy.reshape(N_Q)
    qzf = qz.reshape(N_Q)
    qbf = qb.reshape(N_Q)
    qall = jnp.stack([qxf, qyf, qzf, qbf.astype(jnp.float32)], axis=1)

    rep = lambda a: jnp.broadcast_to(a.reshape(1, N_PTS), (8, N_PTS))
    nbr, d2v = _knn(qall, rep(pos[:, 0]), rep(pos[:, 1]), rep(pos[:, 2]),
                    rep(batch.astype(jnp.float32)))

    table = jnp.concatenate(
        [x, pos, jnp.zeros((N_PTS, D_TAB - 67), jnp.float32)], axis=1)
    idx_flat = nbr.T.reshape(-1)
    g = _sc_gather(table, idx_flat)
    g3 = g.reshape(KMAX, N_Q, D_TAB)

    weights = [
        [(W0_0, b0_0), (W0_1, b0_1), (W0_2, b0_2)],
        [(W1_0, b1_0), (W1_1, b1_1), (W1_2, b1_2)],
        [(W2_0, b2_0), (W2_1, b2_1), (W2_2, b2_2)],
    ]
    outs = []
    for i in range(3):
        (W0, b0), (W1, b1), (W2, b2) = weights[i]
        w0p = jnp.zeros((D_TAB, W0.shape[1]), jnp.float32).at[:67].set(W0)
        params = ((w0p, b0.reshape(1, -1)), (W1, b1.reshape(1, -1)),
                  (W2, b2.reshape(1, -1)))
        outs.append(_conv(g3, qall, d2v, params, _K_LIST[i], _R_LIST[i]))

    new_x = jnp.concatenate(outs, axis=1)
    q_pos = jnp.stack([qxf, qyf, qzf], axis=1)
    return new_x, q_pos, qbf.astype(batch.dtype)
